# baseline (speedup 1.0000x reference)
import functools

import jax
import jax.numpy as jnp
from jax import lax
from jax.experimental import pallas as pl
from jax.experimental.pallas import tpu as pltpu
from jax.experimental.pallas import tpu_sc as plsc

DIM = 64
LANES = 16
NUM_CORES = 2
NUM_SUBCORES = 16
NUM_WORKERS = NUM_CORES * NUM_SUBCORES
CHUNK = 128
HIGH_MASK = jnp.int32(-65536)


def _sc_body(bpw, tw_hbm, cw_hbm, ttp_hbm, ctp_hbm, out_hbm,
             tidx_v, cidx_v, tR_v, cR_v, tbuf_v, cbuf_v, out_v,
             sem_t, sem_c):
    wid = lax.axis_index("s") * NUM_CORES + lax.axis_index("c")
    base = wid * bpw

    pltpu.sync_copy(tw_hbm.at[pl.ds(base, bpw)], tidx_v)
    pltpu.sync_copy(cw_hbm.at[pl.ds(base, bpw)], cidx_v)

    def rbuild(i, carry):
        tR_v[pl.ds(i * LANES, LANES)] = tidx_v[pl.ds(i * LANES, LANES)] >> 2
        cR_v[pl.ds(i * LANES, LANES)] = cidx_v[pl.ds(i * LANES, LANES)] >> 2
        return carry

    lax.fori_loop(0, bpw // LANES, rbuild, 0)

    lane = lax.iota(jnp.int32, LANES)
    lane0 = lane == 0

    def chunk_body(ch, carry):
        cbase = ch * CHUNK
        cp_t = pltpu.async_copy(
            ttp_hbm.at[tR_v.at[pl.ds(cbase, CHUNK)]], tbuf_v, sem_t)
        cp_c = pltpu.async_copy(
            ctp_hbm.at[cR_v.at[pl.ds(cbase, CHUNK)]], cbuf_v, sem_c)
        cp_t.wait()
        cp_c.wait()

        def grp_body(g, carry2):
            vt = tidx_v[pl.ds(cbase + g * LANES, LANES)] & 3
            vc = cidx_v[pl.ds(cbase + g * LANES, LANES)] & 3
            for j in range(LANES):
                slot = g * LANES + j
                ot = vt[j] * 32
                oc = vc[j] * 32
                acc = None
                for k in range(2):
                    at = tbuf_v[slot, pl.ds(ot + k * LANES, LANES)]
                    ac = cbuf_v[slot, pl.ds(oc + k * LANES, LANES)]
                    tlo = plsc.bitcast(at << 16, jnp.float32)
                    thi = plsc.bitcast(at & HIGH_MASK, jnp.float32)
                    clo = plsc.bitcast(ac << 16, jnp.float32)
                    chi = plsc.bitcast(ac & HIGH_MASK, jnp.float32)
                    part = tlo * clo + thi * chi
                    acc = part if acc is None else acc + part
                s = jnp.sum(acc)
                plsc.store_scatter(
                    out_v, [jnp.full((LANES,), cbase + slot, jnp.int32)],
                    jnp.full((LANES,), s, jnp.float32), mask=lane0)
            return carry2

        lax.fori_loop(0, CHUNK // LANES, grp_body, 0)
        return carry

    lax.fori_loop(0, bpw // CHUNK, chunk_body, 0)

    pltpu.sync_copy(out_v, out_hbm.at[pl.ds(base, bpw)])


def _pack_table(table):
    v = table.shape[0]
    half = table.astype(jnp.bfloat16).reshape(v, DIM // 2, 2)
    packed = jax.lax.bitcast_convert_type(half, jnp.int32)
    return packed.reshape(v // 4, 128)


def kernel(target_word, context_words, target_table, context_table):
    b = target_word.shape[0]
    bpw = b // NUM_WORKERS
    mesh = plsc.VectorSubcoreMesh(core_axis_name="c", subcore_axis_name="s")

    sc_call = pl.kernel(
        functools.partial(_sc_body, bpw),
        mesh=mesh,
        compiler_params=pltpu.CompilerParams(
            needs_layout_passes=False, use_tc_tiling_on_sc=True),
        out_type=jax.ShapeDtypeStruct((b,), jnp.float32),
        scratch_types=[
            pltpu.VMEM((bpw,), jnp.int32),
            pltpu.VMEM((bpw,), jnp.int32),
            pltpu.VMEM((bpw,), jnp.int32),
            pltpu.VMEM((bpw,), jnp.int32),
            pltpu.VMEM((CHUNK, 128), jnp.int32),
            pltpu.VMEM((CHUNK, 128), jnp.int32),
            pltpu.VMEM((bpw,), jnp.float32),
            pltpu.SemaphoreType.DMA,
            pltpu.SemaphoreType.DMA,
        ],
    )
    return sc_call(target_word.astype(jnp.int32),
                   context_words.astype(jnp.int32),
                   _pack_table(target_table), _pack_table(context_table))


# speedup vs baseline: 2.8060x; 2.8060x over previous
import functools

import jax
import jax.numpy as jnp
from jax import lax
from jax.experimental import pallas as pl
from jax.experimental.pallas import tpu as pltpu
from jax.experimental.pallas import tpu_sc as plsc

DIM = 64
LANES = 16
NUM_CORES = 2
NUM_SUBCORES = 16
NUM_WORKERS = NUM_CORES * NUM_SUBCORES
CHUNK = 64


def _sc_body(bpw, tw_hbm, cw_hbm, ttp_hbm, ctp_hbm, out_hbm,
             tidx_v, cidx_v, tR_v, cR_v, tbuf_v, cbuf_v, out_v,
             sem_t, sem_c):
    wid = lax.axis_index("s") * NUM_CORES + lax.axis_index("c")
    base = wid * bpw

    pltpu.sync_copy(tw_hbm.at[pl.ds(base, bpw)], tidx_v)
    pltpu.sync_copy(cw_hbm.at[pl.ds(base, bpw)], cidx_v)

    def rbuild(i, carry):
        tR_v[pl.ds(i * LANES, LANES)] = tidx_v[pl.ds(i * LANES, LANES)] >> 2
        cR_v[pl.ds(i * LANES, LANES)] = cidx_v[pl.ds(i * LANES, LANES)] >> 2
        return carry

    lax.fori_loop(0, bpw // LANES, rbuild, 0)

    lane = lax.iota(jnp.int32, LANES)
    lane0 = lane == 0

    def chunk_body(ch, carry):
        cbase = ch * CHUNK
        cp_t = pltpu.async_copy(
            ttp_hbm.at[tR_v.at[pl.ds(cbase, CHUNK)]], tbuf_v, sem_t)
        cp_c = pltpu.async_copy(
            ctp_hbm.at[cR_v.at[pl.ds(cbase, CHUNK)]], cbuf_v, sem_c)
        cp_t.wait()
        cp_c.wait()

        def grp_body(g, carry2):
            vt = (tidx_v[pl.ds(cbase + g * LANES, LANES)] & 3) * DIM
            vc = (cidx_v[pl.ds(cbase + g * LANES, LANES)] & 3) * DIM
            for j in range(LANES):
                slot = g * LANES + j
                ot = vt[j]
                oc = vc[j]
                acc = (tbuf_v[slot, pl.ds(ot, LANES)]
                       * cbuf_v[slot, pl.ds(oc, LANES)])
                for k in range(1, DIM // LANES):
                    acc = acc + (tbuf_v[slot, pl.ds(ot + k * LANES, LANES)]
                                 * cbuf_v[slot, pl.ds(oc + k * LANES, LANES)])
                s = jnp.sum(acc)
                plsc.store_scatter(
                    out_v, [jnp.full((LANES,), cbase + slot, jnp.int32)],
                    jnp.full((LANES,), s, jnp.float32), mask=lane0)
            return carry2

        lax.fori_loop(0, CHUNK // LANES, grp_body, 0)
        return carry

    lax.fori_loop(0, bpw // CHUNK, chunk_body, 0)

    pltpu.sync_copy(out_v, out_hbm.at[pl.ds(base, bpw)])


def kernel(target_word, context_words, target_table, context_table):
    b = target_word.shape[0]
    v = target_table.shape[0]
    bpw = b // NUM_WORKERS
    ttp = target_table.reshape(v // 4, 4 * DIM)
    ctp = context_table.reshape(v // 4, 4 * DIM)
    mesh = plsc.VectorSubcoreMesh(core_axis_name="c", subcore_axis_name="s")

    sc_call = pl.kernel(
        functools.partial(_sc_body, bpw),
        mesh=mesh,
        compiler_params=pltpu.CompilerParams(
            needs_layout_passes=False, use_tc_tiling_on_sc=True),
        out_type=jax.ShapeDtypeStruct((b,), jnp.float32),
        scratch_types=[
            pltpu.VMEM((bpw,), jnp.int32),
            pltpu.VMEM((bpw,), jnp.int32),
            pltpu.VMEM((bpw,), jnp.int32),
            pltpu.VMEM((bpw,), jnp.int32),
            pltpu.VMEM((CHUNK, 4 * DIM), jnp.float32),
            pltpu.VMEM((CHUNK, 4 * DIM), jnp.float32),
            pltpu.VMEM((bpw,), jnp.float32),
            pltpu.SemaphoreType.DMA,
            pltpu.SemaphoreType.DMA,
        ],
    )
    return sc_call(target_word.astype(jnp.int32),
                   context_words.astype(jnp.int32), ttp, ctp)


# speedup vs baseline: 4.5542x; 1.6230x over previous
import functools

import jax
import jax.numpy as jnp
from jax import lax
from jax.experimental import pallas as pl
from jax.experimental.pallas import tpu as pltpu
from jax.experimental.pallas import tpu_sc as plsc

DIM = 64
LANES = 16
NUM_CORES = 2
NUM_SUBCORES = 16
NUM_WORKERS = NUM_CORES * NUM_SUBCORES
GROUP = 16


def _sc_body(bpw, tw_hbm, cw_hbm, tt_hbm, ct_hbm, out_hbm,
             tidx_v, cidx_v, trows_v, crows_v, out_v, sem_a, sem_b):
    wid = lax.axis_index("s") * NUM_CORES + lax.axis_index("c")
    base = wid * bpw

    pltpu.sync_copy(tw_hbm.at[pl.ds(base, bpw)], tidx_v)
    pltpu.sync_copy(cw_hbm.at[pl.ds(base, bpw)], cidx_v)

    lane = lax.iota(jnp.int32, LANES)
    lane0 = lane == 0
    hrows = trows_v.shape[0]
    ngroups = hrows // GROUP

    def fire(gslot, sem, hbase):
        vt = tidx_v[pl.ds(hbase + gslot * GROUP, GROUP)]
        vc = cidx_v[pl.ds(hbase + gslot * GROUP, GROUP)]
        for j in range(GROUP):
            r = gslot * GROUP + j
            pltpu.async_copy(
                tt_hbm.at[pl.ds(vt[j], 1), :],
                trows_v.at[pl.ds(r, 1), :], sem)
            pltpu.async_copy(
                ct_hbm.at[pl.ds(vc[j], 1), :],
                crows_v.at[pl.ds(r, 1), :], sem)

    def drain(sem):
        for j in range(GROUP):
            pltpu.make_async_copy(
                tt_hbm.at[pl.ds(0, 1), :],
                trows_v.at[pl.ds(j, 1), :], sem).wait()
            pltpu.make_async_copy(
                ct_hbm.at[pl.ds(0, 1), :],
                crows_v.at[pl.ds(j, 1), :], sem).wait()

    def half_body(h, carry):
        hbase = h * hrows

        fire(0, sem_a, hbase)

        def fetch_group(g, carry2):
            even = (g % 2) == 0

            @pl.when(jnp.logical_and(g < ngroups, even))
            def _():
                fire(g, sem_a, hbase)

            @pl.when(jnp.logical_and(g < ngroups, jnp.logical_not(even)))
            def _():
                fire(g, sem_b, hbase)

            @pl.when((g % 2) == 1)
            def _():
                drain(sem_a)

            @pl.when((g % 2) == 0)
            def _():
                drain(sem_b)

            return carry2

        lax.fori_loop(1, ngroups + 1, fetch_group, 0)

        def row_body(r, carry2):
            acc = (trows_v[r, pl.ds(0, LANES)] * crows_v[r, pl.ds(0, LANES)])
            for k in range(1, DIM // LANES):
                acc = acc + (trows_v[r, pl.ds(k * LANES, LANES)]
                             * crows_v[r, pl.ds(k * LANES, LANES)])
            s = jnp.sum(acc)
            plsc.store_scatter(out_v,
                               [jnp.full((LANES,), hbase + r, jnp.int32)],
                               jnp.full((LANES,), s, jnp.float32), mask=lane0)
            return carry2

        lax.fori_loop(0, hrows, row_body, 0, unroll=4)
        return carry

    lax.fori_loop(0, bpw // hrows, half_body, 0)

    pltpu.sync_copy(out_v, out_hbm.at[pl.ds(base, bpw)])


def kernel(target_word, context_words, target_table, context_table):
    b = target_word.shape[0]
    bpw = b // NUM_WORKERS
    mesh = plsc.VectorSubcoreMesh(core_axis_name="c", subcore_axis_name="s")

    sc_call = pl.kernel(
        functools.partial(_sc_body, bpw),
        mesh=mesh,
        compiler_params=pltpu.CompilerParams(
            needs_layout_passes=False, use_tc_tiling_on_sc=True),
        out_type=jax.ShapeDtypeStruct((b,), jnp.float32),
        scratch_types=[
            pltpu.VMEM((bpw,), jnp.int32),
            pltpu.VMEM((bpw,), jnp.int32),
            pltpu.VMEM((bpw // 2, DIM), jnp.float32),
            pltpu.VMEM((bpw // 2, DIM), jnp.float32),
            pltpu.VMEM((bpw,), jnp.float32),
            pltpu.SemaphoreType.DMA,
            pltpu.SemaphoreType.DMA,
        ],
    )
    return sc_call(target_word.astype(jnp.int32),
                   context_words.astype(jnp.int32),
                   target_table, context_table)
